# baseline (speedup 1.0000x reference)
import functools

import jax
import jax.numpy as jnp
from jax import lax
from jax.experimental import pallas as pl
from jax.experimental.pallas import tpu as pltpu
from jax.experimental.pallas import tpu_sc as plsc

F = 26
V = 100000
D = 16
B = 16384

NC = 2
NS = 16
NW = NC * NS
L = 16

ROWS = B * F
ROWS_PER_W = ROWS // NW
CHUNK = 128
NCHUNK = ROWS_PER_W // CHUNK
NBUF = 4


def _emb_kernel(idx_hbm, tab_hbm, out_hbm, idx_v, rows_v, gsem):
    wid = lax.axis_index("s") * NC + lax.axis_index("c")
    base = wid * ROWS_PER_W

    pltpu.sync_copy(idx_hbm.at[pl.ds(wid * NCHUNK, NCHUNK)], idx_v)

    lane = lax.iota(jnp.int32, L)

    def offset_body(j, _):
        for k in range(CHUNK // L):
            r0 = base + j * CHUNK + k * L
            f = lax.rem(r0 + lane, F)
            idx_v[j, pl.ds(k * L, L)] = idx_v[j, pl.ds(k * L, L)] + f * V
        return 0

    lax.fori_loop(0, NCHUNK, offset_body, 0)

    def gather_start(j, slot):
        pltpu.async_copy(tab_hbm.at[idx_v.at[j]], rows_v.at[slot], gsem.at[slot])

    def gather_wait(j, slot):
        pltpu.make_async_copy(
            tab_hbm.at[idx_v.at[j]], rows_v.at[slot], gsem.at[slot]
        ).wait()

    for b in range(NBUF):
        gather_start(b, b)

    def super_body(g, _):
        for b in range(NBUF):
            j = g * NBUF + b
            gather_wait(j, b)
            pltpu.sync_copy(
                rows_v.at[b], out_hbm.at[pl.ds(base + j * CHUNK, CHUNK)]
            )
            nxt = j + NBUF

            @pl.when(nxt < NCHUNK)
            def _():
                gather_start(nxt, b)

        return 0

    lax.fori_loop(0, NCHUNK // NBUF, super_body, 0)


@jax.jit
def kernel(indices, tables):
    idx_flat = indices.reshape(ROWS // CHUNK, CHUNK)
    tab_flat = tables.reshape(F * V, D)
    run = functools.partial(
        pl.kernel,
        mesh=plsc.VectorSubcoreMesh(core_axis_name="c", subcore_axis_name="s"),
        out_type=jax.ShapeDtypeStruct((ROWS, D), jnp.float32),
        scratch_types=[
            pltpu.VMEM((NCHUNK, CHUNK), jnp.int32),
            pltpu.VMEM((NBUF, CHUNK, D), jnp.float32),
            pltpu.SemaphoreType.DMA((NBUF,)),
        ],
        compiler_params=pltpu.CompilerParams(use_tc_tiling_on_sc=False),
    )(_emb_kernel)
    out = run(idx_flat, tab_flat)
    return out.reshape(B, F, D)


# speedup vs baseline: 1.1654x; 1.1654x over previous
import functools

import jax
import jax.numpy as jnp
from jax import lax
from jax.experimental import pallas as pl
from jax.experimental.pallas import tpu as pltpu
from jax.experimental.pallas import tpu_sc as plsc

F = 26
V = 100000
D = 16
B = 16384

NC = 2
NS = 16
NW = NC * NS
L = 16

TB = B // NW
SB = 128
NSUP = TB // SB
NHALF = 2


def _emb_kernel(idx_hbm, tab_hbm, out_hbm, idx_v, stage, hsem, osem):
    wid = lax.axis_index("s") * NC + lax.axis_index("c")
    b0 = wid * TB

    pltpu.sync_copy(idx_hbm.at[pl.ds(b0, TB)], idx_v)

    lane = lax.iota(jnp.int32, L)
    off_lo = lane * V
    f_hi = lane + (F - L)
    off_hi = jnp.where(lane >= 2 * L - F, f_hi * V, 0)

    def off_body(b, _):
        idx_v[b, pl.ds(0, L)] = idx_v[b, pl.ds(0, L)] + off_lo
        idx_v[b, pl.ds(F - L, L)] = idx_v[b, pl.ds(F - L, L)] + off_hi
        return 0

    lax.fori_loop(0, TB, off_body, 0)

    def fill(g, s):
        def fire(k, _):
            pltpu.async_copy(
                tab_hbm.at[idx_v.at[g * SB + k]], stage.at[s, k], hsem.at[s]
            )
            return 0

        lax.fori_loop(0, SB, fire, 0)

        def drain(k, _):
            pltpu.make_async_copy(
                tab_hbm.at[idx_v.at[g * SB + k]], stage.at[s, k], hsem.at[s]
            ).wait()
            return 0

        lax.fori_loop(0, SB, drain, 0)

    def write_start(g, s):
        pltpu.async_copy(
            stage.at[s], out_hbm.at[pl.ds(b0 + g * SB, SB)], osem.at[s]
        )

    def write_wait(g, s):
        pltpu.make_async_copy(
            stage.at[s], out_hbm.at[pl.ds(b0 + g * SB, SB)], osem.at[s]
        ).wait()

    fill(0, 0)

    def super_body(g, _):
        s = lax.rem(g, NHALF)
        write_start(g, s)
        nxt = g + 1

        @pl.when(nxt < NSUP)
        def _():
            sn = lax.rem(nxt, NHALF)

            @pl.when(nxt >= NHALF)
            def _():
                write_wait(nxt - NHALF, sn)

            fill(nxt, sn)

        return 0

    lax.fori_loop(0, NSUP, super_body, 0)

    write_wait(NSUP - 2, NSUP % NHALF)
    write_wait(NSUP - 1, (NSUP - 1) % NHALF)


@jax.jit
def kernel(indices, tables):
    tab_flat = tables.reshape(F * V, D)
    run = functools.partial(
        pl.kernel,
        mesh=plsc.VectorSubcoreMesh(core_axis_name="c", subcore_axis_name="s"),
        out_type=jax.ShapeDtypeStruct((B, F, D), jnp.float32),
        scratch_types=[
            pltpu.VMEM((TB, F), jnp.int32),
            pltpu.VMEM((NHALF, SB, F, D), jnp.float32),
            pltpu.SemaphoreType.DMA((NHALF,)),
            pltpu.SemaphoreType.DMA((NHALF,)),
        ],
        compiler_params=pltpu.CompilerParams(use_tc_tiling_on_sc=False),
    )(_emb_kernel)
    return run(indices, tab_flat)
